# Initial kernel scaffold; baseline (speedup 1.0000x reference)
import jax
import jax.numpy as jnp
from jax.experimental import pallas as pl


def kernel(dense_features, sharded_sparse_features, tables, w1, b1, w2, b2, w3, b3, w4, b4):
    raise NotImplementedError("write your pallas kernel here")


# speedup vs baseline: 2.2033x; 2.2033x over previous
import functools
import jax
import jax.numpy as jnp
from jax import lax
from jax.experimental import pallas as pl
from jax.experimental.pallas import tpu as pltpu
from jax.experimental.pallas import tpu_sc as plsc

_B = 4096
_T = 26
_V = 100000
_D = 32
_DF = 128
_H = 512

_NC = 2
_NS = 16
_NW = _NC * _NS
_TPW = _B * _T // _NW
_GCH = 128
_NCHUNK = _TPW // _GCH


def _sc_gather_body(tab_hbm, idx_hbm, off_hbm, out_hbm, idx_v, off_v, rows_v, sem):
    c = lax.axis_index("c")
    s = lax.axis_index("s")
    wid = s * _NC + c
    base = wid * _TPW

    pltpu.sync_copy(idx_hbm.at[pl.ds(base, _TPW)], idx_v)
    pltpu.sync_copy(off_hbm, off_v)

    def add_off(k, _):
        sl = pl.ds(k * 16, 16)
        idx_v[sl] = idx_v[sl] + off_v[sl]
        return 0

    lax.fori_loop(0, _TPW // 16, add_off, 0)

    def fire(j, _):
        sl = pl.ds(j * _GCH, _GCH)
        pltpu.async_copy(tab_hbm.at[idx_v.at[sl]], rows_v.at[sl], sem)
        return 0

    lax.fori_loop(0, _NCHUNK, fire, 0)

    pltpu.make_async_copy(out_hbm.at[pl.ds(base, _TPW)], rows_v, sem).wait()

    pltpu.sync_copy(rows_v, out_hbm.at[pl.ds(base, _TPW)])


@jax.jit
def _sc_gather(tab, idx, off):
    mesh = plsc.VectorSubcoreMesh(core_axis_name="c", subcore_axis_name="s")
    return pl.kernel(
        _sc_gather_body,
        out_type=jax.ShapeDtypeStruct((_B * _T, _D), jnp.float32),
        mesh=mesh,
        scratch_types=[
            pltpu.VMEM((_TPW,), jnp.int32),
            pltpu.VMEM((_TPW,), jnp.int32),
            pltpu.VMEM((_TPW, _D), jnp.float32),
            pltpu.SemaphoreType.DMA,
        ],
        compiler_params=pltpu.CompilerParams(use_tc_tiling_on_sc=False),
    )(tab, idx, off)


def _mlp_body(df_ref, emb_ref, w1_ref, b1_ref, w2_ref, b2_ref, w3_ref, b3_ref,
              w4_ref, b4_ref, out_ref):
    f32 = jnp.float32
    h = jnp.maximum(
        jnp.dot(df_ref[...], w1_ref[...], preferred_element_type=f32) + b1_ref[...], 0.0)
    dx = jnp.maximum(
        jnp.dot(h, w2_ref[...], preferred_element_type=f32) + b2_ref[...], 0.0)
    g = (jnp.dot(dx, w3_ref[0:_D, :], preferred_element_type=f32)
         + jnp.dot(emb_ref[...], w3_ref[_D:, :], preferred_element_type=f32)
         + b3_ref[...])
    g = jnp.maximum(g, 0.0)
    out_ref[...] = jnp.maximum(
        jnp.dot(g, w4_ref[...], preferred_element_type=f32) + b4_ref[...], 0.0)


@functools.partial(jax.jit, static_argnames=("bb",))
def _tc_mlp(df, emb2, w1, b1, w2, b2, w3, b3, w4, b4, bb=512):
    grid = (_B // bb,)
    full = lambda shape: pl.BlockSpec(shape, lambda i: (0, 0))
    return pl.pallas_call(
        _mlp_body,
        grid=grid,
        in_specs=[
            pl.BlockSpec((bb, _DF), lambda i: (i, 0)),
            pl.BlockSpec((bb, _T * _D), lambda i: (i, 0)),
            full((_DF, _DF)),
            full((1, _DF)),
            full((_DF, _D)),
            full((1, _D)),
            full((_D + _T * _D, _H)),
            full((1, _H)),
            full((_H, 1)),
            full((1, 1)),
        ],
        out_specs=pl.BlockSpec((bb, 1), lambda i: (i, 0)),
        out_shape=jax.ShapeDtypeStruct((_B, 1), jnp.float32),
        compiler_params=pltpu.CompilerParams(
            dimension_semantics=("arbitrary",),
        ),
    )(df, emb2, w1, b1, w2, b2, w3, b3, w4, b4)


def kernel(dense_features, sharded_sparse_features, tables, w1, b1, w2, b2, w3, b3, w4, b4):
    idx = sharded_sparse_features.astype(jnp.int32).reshape(_B * _T)
    tab = tables.reshape(_T * _V, _D)
    off = jnp.tile(jnp.arange(_T, dtype=jnp.int32) * _V, _TPW // _T)
    emb_flat = _sc_gather(tab, idx, off)
    emb2 = emb_flat.reshape(_B, _T * _D)
    return _tc_mlp(
        dense_features, emb2,
        w1, b1.reshape(1, _DF),
        w2, b2.reshape(1, _D),
        w3, b3.reshape(1, _H),
        w4, b4.reshape(1, 1),
    )


# speedup vs baseline: 10.9384x; 4.9645x over previous
import functools
import jax
import jax.numpy as jnp
from jax import lax
from jax.experimental import pallas as pl
from jax.experimental.pallas import tpu as pltpu
from jax.experimental.pallas import tpu_sc as plsc

_B = 4096
_T = 26
_V = 100000
_D = 32
_DF = 128
_H = 512

_NC = 2
_NS = 16
_NW = _NC * _NS


def _sc_gather_body(tab_hbm, idx_hbm, out_hbm, buf_v, idx_v, out_v, semt, semi):
    c = lax.axis_index("c")
    s = lax.axis_index("s")
    w = s * _NC + c

    def per_table(t, _):
        cpt = pltpu.make_async_copy(tab_hbm.at[t, w], buf_v, semt)
        cpt.start()
        cpi = pltpu.make_async_copy(idx_hbm.at[t], idx_v, semi)
        cpi.start()
        cpi.wait()
        cpt.wait()

        def per_vreg(k, _):
            sl = pl.ds(k * 16, 16)
            out_v[sl] = plsc.load_gather(buf_v, [idx_v[sl]])
            return 0

        lax.fori_loop(0, _B // 16, per_vreg, 0)
        pltpu.sync_copy(out_v, out_hbm.at[t * _D + w])
        return 0

    lax.fori_loop(0, _T, per_table, 0)


@jax.jit
def _sc_gather(tab, idx_t):
    mesh = plsc.VectorSubcoreMesh(core_axis_name="c", subcore_axis_name="s")
    return pl.kernel(
        _sc_gather_body,
        out_type=jax.ShapeDtypeStruct((_T * _D, _B), jnp.float32),
        mesh=mesh,
        scratch_types=[
            pltpu.VMEM((_V,), jnp.float32),
            pltpu.VMEM((_B,), jnp.int32),
            pltpu.VMEM((_B,), jnp.float32),
            pltpu.SemaphoreType.DMA,
            pltpu.SemaphoreType.DMA,
        ],
        compiler_params=pltpu.CompilerParams(
            use_tc_tiling_on_sc=True, needs_layout_passes=False),
    )(tab, idx_t)


def _mlp_body(df_ref, embt_ref, w1_ref, b1_ref, w2_ref, b2_ref, w3_ref, b3_ref,
              w4_ref, b4_ref, out_ref):
    f32 = jnp.float32
    h = jnp.maximum(
        jnp.dot(df_ref[...], w1_ref[...], preferred_element_type=f32) + b1_ref[...], 0.0)
    dx = jnp.maximum(
        jnp.dot(h, w2_ref[...], preferred_element_type=f32) + b2_ref[...], 0.0)
    emb_w3 = lax.dot_general(
        embt_ref[...], w3_ref[_D:, :],
        dimension_numbers=(((0,), (0,)), ((), ())),
        preferred_element_type=f32)
    g = (jnp.dot(dx, w3_ref[0:_D, :], preferred_element_type=f32)
         + emb_w3 + b3_ref[...])
    g = jnp.maximum(g, 0.0)
    out_ref[...] = jnp.maximum(
        jnp.dot(g, w4_ref[...], preferred_element_type=f32) + b4_ref[...], 0.0)


@functools.partial(jax.jit, static_argnames=("bb",))
def _tc_mlp(df, embt, w1, b1, w2, b2, w3, b3, w4, b4, bb=512):
    grid = (_B // bb,)
    full = lambda shape: pl.BlockSpec(shape, lambda i: (0, 0))
    return pl.pallas_call(
        _mlp_body,
        grid=grid,
        in_specs=[
            pl.BlockSpec((bb, _DF), lambda i: (i, 0)),
            pl.BlockSpec((_T * _D, bb), lambda i: (0, i)),
            full((_DF, _DF)),
            full((1, _DF)),
            full((_DF, _D)),
            full((1, _D)),
            full((_D + _T * _D, _H)),
            full((1, _H)),
            full((_H, 1)),
            full((1, 1)),
        ],
        out_specs=pl.BlockSpec((bb, 1), lambda i: (i, 0)),
        out_shape=jax.ShapeDtypeStruct((_B, 1), jnp.float32),
        compiler_params=pltpu.CompilerParams(
            dimension_semantics=("arbitrary",),
        ),
    )(df, embt, w1, b1, w2, b2, w3, b3, w4, b4)


def kernel(dense_features, sharded_sparse_features, tables, w1, b1, w2, b2, w3, b3, w4, b4):
    tt = jnp.transpose(tables, (0, 2, 1))
    idx_t = sharded_sparse_features.astype(jnp.int32).T
    embt = _sc_gather(tt, idx_t)
    return _tc_mlp(
        dense_features, embt,
        w1, b1.reshape(1, _DF),
        w2, b2.reshape(1, _D),
        w3, b3.reshape(1, _H),
        w4, b4.reshape(1, 1),
    )
